# baseline (speedup 1.0000x reference)
import jax
import jax.numpy as jnp
from jax.experimental import pallas as pl
from jax.experimental.pallas import tpu as pltpu


_CHUNK_ROWS = 512
_NBUF = 4


def _copy_body(w_ref, o_ref, buf, load_sems, store_sems):
    rows = w_ref.shape[0]
    n = rows // _CHUNK_ROWS

    def load(i, b):
        return pltpu.make_async_copy(
            w_ref.at[pl.ds(i * _CHUNK_ROWS, _CHUNK_ROWS)],
            buf.at[b],
            load_sems.at[b],
        )

    def store(i, b):
        return pltpu.make_async_copy(
            buf.at[b],
            o_ref.at[pl.ds(i * _CHUNK_ROWS, _CHUNK_ROWS)],
            store_sems.at[b],
        )

    for k in range(min(_NBUF, n)):
        load(k, k).start()
    for i in range(n):
        b = i % _NBUF
        load(i, b).wait()
        store(i, b).start()
        j = i + _NBUF
        if j < n:
            store(i, b).wait()
            load(j, b).start()
    for i in range(max(0, n - _NBUF), n):
        store(i, i % _NBUF).wait()


def kernel(x, emb_weight):
    rows, dim = emb_weight.shape
    assert x.shape[1] == rows and rows % _CHUNK_ROWS == 0
    return pl.pallas_call(
        _copy_body,
        in_specs=[pl.BlockSpec(memory_space=pl.ANY)],
        out_specs=pl.BlockSpec(memory_space=pl.ANY),
        out_shape=jax.ShapeDtypeStruct((rows, dim), emb_weight.dtype),
        scratch_shapes=[
            pltpu.VMEM((_NBUF, _CHUNK_ROWS, dim), emb_weight.dtype),
            pltpu.SemaphoreType.DMA((_NBUF,)),
            pltpu.SemaphoreType.DMA((_NBUF,)),
        ],
    )(emb_weight)
